# baseline (speedup 1.0000x reference)
import functools

import jax
import jax.numpy as jnp
from jax import lax
from jax.experimental import pallas as pl
from jax.experimental.pallas import tpu as pltpu
from jax.experimental.pallas import tpu_sc as plsc

B = 4096
L = 200
D = 64
NC = 2
NS = 16
NW = NC * NS
TOTAL = B * L
PER_TILE = TOTAL // NW
IDX_MINOR = 128
CHUNK = 512
IDX_ROWS = CHUNK // IDX_MINOR
STAGES = PER_TILE // CHUNK
TILE_IDX_ROWS = PER_TILE // IDX_MINOR


def _gather_body(idx_hbm, table_hbm, out_hbm, idx_all, rows0, rows1,
                 gsem0, gsem1, ssem0, ssem1):
    c = lax.axis_index("c")
    s = lax.axis_index("s")
    wid = s * NC + c
    idx_base = wid * TILE_IDX_ROWS
    row_base = wid * PER_TILE

    rows = (rows0, rows1)
    gsem = (gsem0, gsem1)
    ssem = (ssem0, ssem1)

    pltpu.sync_copy(idx_hbm.at[pl.ds(idx_base, TILE_IDX_ROWS)], idx_all)

    def fire_gather(g, b):
        for j in range(IDX_ROWS):
            pltpu.async_copy(table_hbm.at[idx_all.at[g * IDX_ROWS + j]],
                             rows[b].at[pl.ds(j * IDX_MINOR, IDX_MINOR)],
                             gsem[b])

    def wait_gather(b):
        pltpu.make_async_copy(out_hbm.at[pl.ds(row_base, CHUNK)],
                              rows[b], gsem[b]).wait()

    def fire_store(g, b):
        pltpu.async_copy(rows[b],
                         out_hbm.at[pl.ds(row_base + g * CHUNK, CHUNK)],
                         ssem[b])

    def wait_store(g, b):
        pltpu.make_async_copy(rows[b],
                              out_hbm.at[pl.ds(row_base + g * CHUNK, CHUNK)],
                              ssem[b]).wait()

    fire_gather(0, 0)
    fire_gather(1, 1)
    wait_gather(0)
    fire_store(0, 0)

    def pair(k, carry):
        g2 = 2 + 2 * k
        for b in range(2):
            g = g2 + b
            pb = 1 - b
            wait_store(g - 2, b)
            fire_gather(g, b)
            wait_gather(pb)
            fire_store(g - 1, pb)
        return carry

    lax.fori_loop(0, (STAGES - 2) // 2, pair, 0)

    wait_gather(1)
    fire_store(STAGES - 1, 1)
    wait_store(STAGES - 2, 0)
    wait_store(STAGES - 1, 1)


def _sc_gather(idx2d, table):
    mesh = plsc.VectorSubcoreMesh(core_axis_name="c", subcore_axis_name="s",
                                  num_cores=NC, num_subcores=NS)
    fn = pl.kernel(
        _gather_body,
        out_type=jax.ShapeDtypeStruct((TOTAL, D), jnp.float32),
        mesh=mesh,
        scratch_types=[
            pltpu.VMEM((TILE_IDX_ROWS, IDX_MINOR), jnp.int32),
            pltpu.VMEM((CHUNK, D), jnp.float32),
            pltpu.VMEM((CHUNK, D), jnp.float32),
            pltpu.SemaphoreType.DMA,
            pltpu.SemaphoreType.DMA,
            pltpu.SemaphoreType.DMA,
            pltpu.SemaphoreType.DMA,
        ],
        compiler_params=pltpu.CompilerParams(use_tc_tiling_on_sc=False),
    )
    return fn(idx2d, table)


PBLK = 2048


def _price_body(p_ref, w_ref, b_ref, o_ref):
    o_ref[...] = p_ref[...] * w_ref[...] + b_ref[...]


def _price_emb(price_flat, W, b):
    grid = (TOTAL // PBLK,)
    return pl.pallas_call(
        _price_body,
        grid=grid,
        in_specs=[
            pl.BlockSpec((PBLK, 1), lambda i: (i, 0)),
            pl.BlockSpec((1, D), lambda i: (0, 0)),
            pl.BlockSpec((1, D), lambda i: (0, 0)),
        ],
        out_specs=pl.BlockSpec((PBLK, D), lambda i: (i, 0)),
        out_shape=jax.ShapeDtypeStruct((TOTAL, D), jnp.float32),
    )(price_flat, W, b)


@jax.jit
def kernel(item_id, price, emb_table, W, b):
    idx2d = item_id.reshape(TOTAL // IDX_MINOR, IDX_MINOR)
    item_emb = _sc_gather(idx2d, emb_table).reshape(B, L, D)
    price_flat = price.reshape(TOTAL, 1)
    price_emb = _price_emb(price_flat, W, b.reshape(1, D)).reshape(B, L, D)
    return (item_emb, price_emb)


# speedup vs baseline: 1.1694x; 1.1694x over previous
import functools

import jax
import jax.numpy as jnp
from jax import lax
from jax.experimental import pallas as pl
from jax.experimental.pallas import tpu as pltpu
from jax.experimental.pallas import tpu_sc as plsc

B = 4096
L = 200
D = 64
NC = 2
NS = 16
NW = NC * NS
TOTAL = B * L
PER_TILE = TOTAL // NW
IDX_MINOR = 128
CHUNK = 256
IDX_ROWS = CHUNK // IDX_MINOR
STAGES = PER_TILE // CHUNK
TILE_IDX_ROWS = PER_TILE // IDX_MINOR
NBUF = 4


def _gather_body(idx_hbm, table_hbm, out_hbm, idx_all,
                 rows0, rows1, rows2, rows3,
                 gsem0, gsem1, gsem2, gsem3,
                 ssem0, ssem1, ssem2, ssem3):
    c = lax.axis_index("c")
    s = lax.axis_index("s")
    wid = s * NC + c
    idx_base = wid * TILE_IDX_ROWS
    row_base = wid * PER_TILE

    rows = (rows0, rows1, rows2, rows3)
    gsem = (gsem0, gsem1, gsem2, gsem3)
    ssem = (ssem0, ssem1, ssem2, ssem3)

    pltpu.sync_copy(idx_hbm.at[pl.ds(idx_base, TILE_IDX_ROWS)], idx_all)

    def fire_gather(g, b):
        for j in range(IDX_ROWS):
            pltpu.async_copy(table_hbm.at[idx_all.at[g * IDX_ROWS + j]],
                             rows[b].at[pl.ds(j * IDX_MINOR, IDX_MINOR)],
                             gsem[b])

    def wait_gather(b):
        for j in range(IDX_ROWS):
            pltpu.make_async_copy(table_hbm.at[idx_all.at[0]],
                                  rows[b].at[pl.ds(j * IDX_MINOR, IDX_MINOR)],
                                  gsem[b]).wait()

    def fire_store(g, b):
        pltpu.async_copy(rows[b],
                         out_hbm.at[pl.ds(row_base + g * CHUNK, CHUNK)],
                         ssem[b])

    def wait_store(g, b):
        pltpu.make_async_copy(rows[b],
                              out_hbm.at[pl.ds(row_base + g * CHUNK, CHUNK)],
                              ssem[b]).wait()

    fire_gather(0, 0)
    fire_gather(1, 1)
    fire_gather(2, 2)
    fire_gather(3, 3)
    wait_gather(0)
    fire_store(0, 0)

    def quad(k, carry):
        g4 = 4 + 4 * k
        for b in range(NBUF):
            g = g4 + b
            wait_store(g - NBUF, b)
            fire_gather(g, b)
            pb = (b + 1) % NBUF
            wait_gather(pb)
            fire_store(g - 3, pb)
        return carry

    lax.fori_loop(0, (STAGES - NBUF) // NBUF, quad, 0)

    for t in (STAGES - 3, STAGES - 2, STAGES - 1):
        bt = t % NBUF
        wait_gather(bt)
        fire_store(t, bt)
    for t in (STAGES - 4, STAGES - 3, STAGES - 2, STAGES - 1):
        wait_store(t, t % NBUF)


def _sc_gather(idx2d, table):
    mesh = plsc.VectorSubcoreMesh(core_axis_name="c", subcore_axis_name="s",
                                  num_cores=NC, num_subcores=NS)
    fn = pl.kernel(
        _gather_body,
        out_type=jax.ShapeDtypeStruct((TOTAL, D), jnp.float32),
        mesh=mesh,
        scratch_types=[
            pltpu.VMEM((TILE_IDX_ROWS, IDX_MINOR), jnp.int32),
            pltpu.VMEM((CHUNK, D), jnp.float32),
            pltpu.VMEM((CHUNK, D), jnp.float32),
            pltpu.VMEM((CHUNK, D), jnp.float32),
            pltpu.VMEM((CHUNK, D), jnp.float32),
            pltpu.SemaphoreType.DMA,
            pltpu.SemaphoreType.DMA,
            pltpu.SemaphoreType.DMA,
            pltpu.SemaphoreType.DMA,
            pltpu.SemaphoreType.DMA,
            pltpu.SemaphoreType.DMA,
            pltpu.SemaphoreType.DMA,
            pltpu.SemaphoreType.DMA,
        ],
        compiler_params=pltpu.CompilerParams(use_tc_tiling_on_sc=False),
    )
    return fn(idx2d, table)


PB = 32


def _price_body(p_ref, w_ref, b_ref, o_ref):
    o_ref[...] = (p_ref[...][:, :, None] * w_ref[...][None, :, :]
                  + b_ref[...][None, :, :])


def _price_emb(price, W, b):
    grid = (B // PB,)
    return pl.pallas_call(
        _price_body,
        grid=grid,
        in_specs=[
            pl.BlockSpec((PB, L), lambda i: (i, 0)),
            pl.BlockSpec((1, D), lambda i: (0, 0)),
            pl.BlockSpec((1, D), lambda i: (0, 0)),
        ],
        out_specs=pl.BlockSpec((PB, L, D), lambda i: (i, 0, 0)),
        out_shape=jax.ShapeDtypeStruct((B, L, D), jnp.float32),
    )(price, W, b)


@jax.jit
def kernel(item_id, price, emb_table, W, b):
    idx2d = item_id.reshape(TOTAL // IDX_MINOR, IDX_MINOR)
    item_emb = _sc_gather(idx2d, emb_table).reshape(B, L, D)
    price_emb = _price_emb(price, W, b.reshape(1, D))
    return (item_emb, price_emb)
